# baseline (speedup 1.0000x reference)
import math

import jax
import jax.numpy as jnp
from jax import lax
from jax.experimental import pallas as pl
from jax.experimental.pallas import tpu as pltpu
from jax.experimental.pallas import tpu_sc as plsc

VOCAB = 1000000
D = 64
L = 16
NC, NS = 2, 16
NW = NC * NS
BATCH = 4096
HIST = 200
ROWS_W = BATCH // NW

MAX_NORM = (1.0 - 0.001) / math.sqrt(1.0)
INV_MAX_NORM = 1.0 / MAX_NORM


def _rsqrt_nr(s):
    i = lax.bitcast_convert_type(s, jnp.int32)
    i = jnp.int32(0x5F3759DF) - lax.shift_right_arithmetic(i, 1)
    y = lax.bitcast_convert_type(i, jnp.float32)
    y = y * (1.5 - 0.5 * (s * y) * y)
    y = y * (1.5 - 0.5 * (s * y) * y)
    y = y * (1.5 - 0.5 * (s * y) * y)
    return y


def _recip_nr(d):
    i = lax.bitcast_convert_type(d, jnp.int32)
    z = lax.bitcast_convert_type(jnp.int32(0x7EF127EA) - i, jnp.float32)
    z = z * (2.0 - d * z)
    z = z * (2.0 - d * z)
    z = z * (2.0 - d * z)
    return z


def _body(ids_hbm, weight_hbm, out_hbm, idx_v, rows_v, fac_v, sem):
    wid = lax.axis_index("s") * NC + lax.axis_index("c")
    base_row = wid * ROWS_W

    def chunk_body(c, _):
        row = base_row + c
        pltpu.sync_copy(ids_hbm.at[row], idx_v)
        pltpu.async_copy(weight_hbm.at[idx_v], rows_v, sem).wait()

        def ss_body(r, _):
            ss = jnp.zeros((L,), jnp.float32)
            for k in range(D // L):
                v = rows_v[r, pl.ds(k * L, L)]
                ss = ss + v * v
            s = jnp.sum(ss)
            rs = _rsqrt_nr(s)
            norm = s * rs
            scale = jnp.minimum(norm * INV_MAX_NORM, 1.0)
            fac_v[r] = _recip_nr(scale + 1e-8)
            return 0

        lax.fori_loop(0, HIST, ss_body, 0, unroll=4)

        def row_body(r, _):
            f = fac_v[r]
            for k in range(D // L):
                rows_v[r, pl.ds(k * L, L)] = rows_v[r, pl.ds(k * L, L)] * f
            return 0

        lax.fori_loop(0, HIST, row_body, 0, unroll=4)
        pltpu.sync_copy(rows_v, out_hbm.at[row])
        return 0

    lax.fori_loop(0, ROWS_W, chunk_body, 0)


@jax.jit
def _run(input_ids, weight):
    mesh = plsc.VectorSubcoreMesh(core_axis_name="c", subcore_axis_name="s")
    return pl.kernel(
        _body,
        out_type=jax.ShapeDtypeStruct((BATCH, HIST, D), jnp.float32),
        mesh=mesh,
        compiler_params=pltpu.CompilerParams(
            needs_layout_passes=False, use_tc_tiling_on_sc=False
        ),
        scratch_types=[
            pltpu.VMEM((HIST,), jnp.int32),
            pltpu.VMEM((HIST, D), jnp.float32),
            pltpu.SMEM((HIST,), jnp.float32),
            pltpu.SemaphoreType.DMA,
        ],
    )(input_ids, weight)


def kernel(input_ids, weight):
    return _run(input_ids, weight)
